# baseline (speedup 1.0000x reference)
import jax
import jax.numpy as jnp
from jax.experimental import pallas as pl
from jax.experimental.pallas import tpu as pltpu

_BM = 400


def _sanbet_kernel(w_ref, b_ref, adj_ref, inp_ref, out_ref):
    a = adj_ref[...].astype(jnp.bfloat16)
    acc = jax.lax.dot_general(
        a, inp_ref[...], (((1,), (0,)), ((), ())),
        preferred_element_type=jnp.float32,
    )
    out_ref[...] = acc * w_ref[0, 0] + b_ref[0, 0]


def kernel(inp, adj, weight, bias):
    n, d = inp.shape
    w2 = weight.reshape(1, 1)
    b2 = bias.reshape(1, 1)
    inp_bf = inp.astype(jnp.bfloat16)
    grid = (n // _BM,)
    return pl.pallas_call(
        _sanbet_kernel,
        grid=grid,
        in_specs=[
            pl.BlockSpec((1, 1), lambda i: (0, 0)),
            pl.BlockSpec((1, 1), lambda i: (0, 0)),
            pl.BlockSpec((_BM, n), lambda i: (i, 0)),
            pl.BlockSpec((n, d), lambda i: (0, 0)),
        ],
        out_specs=pl.BlockSpec((_BM, d), lambda i: (i, 0)),
        out_shape=jax.ShapeDtypeStruct((n, d), jnp.float32),
        compiler_params=pltpu.CompilerParams(
            dimension_semantics=("arbitrary",),
        ),
    )(w2, b2, adj, inp_bf)


# speedup vs baseline: 1.0229x; 1.0229x over previous
import jax
import jax.numpy as jnp
from jax.experimental import pallas as pl
from jax.experimental.pallas import tpu as pltpu

_BM = 400


def _sanbet_kernel(w_ref, b_ref, adj_ref, inp_ref, out_ref):
    acc = jax.lax.dot_general(
        adj_ref[...], inp_ref[...], (((1,), (0,)), ((), ())),
        preferred_element_type=jnp.float32,
        precision=jax.lax.Precision.DEFAULT,
    )
    out_ref[...] = acc * w_ref[0, 0] + b_ref[0, 0]


def kernel(inp, adj, weight, bias):
    n, d = inp.shape
    w2 = weight.reshape(1, 1)
    b2 = bias.reshape(1, 1)
    grid = (n // _BM,)
    return pl.pallas_call(
        _sanbet_kernel,
        grid=grid,
        in_specs=[
            pl.BlockSpec((1, 1), lambda i: (0, 0)),
            pl.BlockSpec((1, 1), lambda i: (0, 0)),
            pl.BlockSpec((_BM, n), lambda i: (i, 0)),
            pl.BlockSpec((n, d), lambda i: (0, 0)),
        ],
        out_specs=pl.BlockSpec((_BM, d), lambda i: (i, 0)),
        out_shape=jax.ShapeDtypeStruct((n, d), jnp.float32),
        compiler_params=pltpu.CompilerParams(
            dimension_semantics=("arbitrary",),
        ),
    )(w2, b2, adj, inp)


# speedup vs baseline: 1.0277x; 1.0047x over previous
import jax
import jax.numpy as jnp
from jax.experimental import pallas as pl
from jax.experimental.pallas import tpu as pltpu

_BM = 200


def _sanbet_kernel(w_ref, b_ref, adj_ref, inp_ref, out_ref):
    acc = jax.lax.dot_general(
        adj_ref[...], inp_ref[...], (((1,), (0,)), ((), ())),
        preferred_element_type=jnp.float32,
        precision=jax.lax.Precision.DEFAULT,
    )
    out_ref[...] = acc * w_ref[0, 0] + b_ref[0, 0]


def kernel(inp, adj, weight, bias):
    n, d = inp.shape
    w2 = weight.reshape(1, 1)
    b2 = bias.reshape(1, 1)
    grid = (n // _BM,)
    return pl.pallas_call(
        _sanbet_kernel,
        grid=grid,
        in_specs=[
            pl.BlockSpec((1, 1), lambda i: (0, 0)),
            pl.BlockSpec((1, 1), lambda i: (0, 0)),
            pl.BlockSpec((_BM, n), lambda i: (i, 0)),
            pl.BlockSpec((n, d), lambda i: (0, 0)),
        ],
        out_specs=pl.BlockSpec((_BM, d), lambda i: (i, 0)),
        out_shape=jax.ShapeDtypeStruct((n, d), jnp.float32),
        compiler_params=pltpu.CompilerParams(
            dimension_semantics=("arbitrary",),
        ),
    )(w2, b2, adj, inp)
